# baseline (speedup 1.0000x reference)
import functools

import jax
import jax.numpy as jnp
from jax import lax
from jax.experimental import pallas as pl
from jax.experimental.pallas import tpu as pltpu
from jax.experimental.pallas import tpu_sc as plsc

N = 10000
E = 320000
D_IN = 128
D_HID = 64
D_OUT = 64
DURATION = 3
CORE_NUM = 3
DIFF_NUM = 2
H3 = 3 * D_OUT

NC = 2
NS = 16
NW = NC * NS
EPW = E // NW
CHUNK = 128
NCH = 80
EPW_P = NCH * CHUNK


def _sc_spmm_body(x_hbm, src_hbm, dst_hbm, w_hbm, zeros_hbm, out_hbm,
                  src_v, dst_v, w_v, rows0, rows1, slab, sem0, sem1):
    ci = lax.axis_index("c")
    si = lax.axis_index("s")
    wid = ci * NS + si
    stripe = pl.ds(si * (N // NS), N // NS)

    def scale(rows_v, k):
        kvec = jnp.broadcast_to(k, (16,)).astype(jnp.int32)
        for i in range(CHUNK):
            wspl = plsc.load_gather(w_v, [kvec, jnp.full((16,), i, jnp.int32)])
            for r in range(4):
                rows_v[i, pl.ds(r * 16, 16)] = rows_v[i, pl.ds(r * 16, 16)] * wspl

    for cc in range(CORE_NUM):
        pltpu.sync_copy(zeros_hbm, slab.at[stripe])
        pltpu.sync_copy(src_hbm.at[cc, wid], src_v)
        pltpu.sync_copy(dst_hbm.at[cc, wid], dst_v)
        pltpu.sync_copy(w_hbm.at[cc, wid], w_v)
        plsc.subcore_barrier()

        pltpu.async_copy(x_hbm.at[src_v.at[0]], rows0, sem0)

        def chunk_body(k2, carry):
            k = 2 * k2
            pltpu.make_async_copy(x_hbm.at[src_v.at[k]], rows0, sem0).wait()
            pltpu.async_copy(x_hbm.at[src_v.at[k + 1]], rows1, sem1)
            scale(rows0, k)
            pltpu.sync_copy(rows0, slab.at[dst_v.at[k]], add=True)

            pltpu.make_async_copy(x_hbm.at[src_v.at[k + 1]], rows1, sem1).wait()

            @pl.when(k + 2 < NCH)
            def _():
                pltpu.async_copy(x_hbm.at[src_v.at[k + 2]], rows0, sem0)

            scale(rows1, k + 1)
            pltpu.sync_copy(rows1, slab.at[dst_v.at[k + 1]], add=True)
            return carry

        lax.fori_loop(0, NCH // 2, chunk_body, 0)

        plsc.subcore_barrier()

        pltpu.sync_copy(slab.at[stripe], out_hbm.at[ci, cc, si])
        plsc.subcore_barrier()


_sc_spmm = pl.kernel(
    _sc_spmm_body,
    out_type=jax.ShapeDtypeStruct((NC, CORE_NUM, NS, N // NS, D_OUT),
                                  jnp.float32),
    mesh=plsc.VectorSubcoreMesh(core_axis_name="c", subcore_axis_name="s",
                                num_cores=NC, num_subcores=NS),
    compiler_params=pltpu.CompilerParams(needs_layout_passes=False,
                                         use_tc_tiling_on_sc=False),
    scratch_types=[
        pltpu.VMEM((NCH, CHUNK), jnp.int32),
        pltpu.VMEM((NCH, CHUNK), jnp.int32),
        pltpu.VMEM((NCH, CHUNK), jnp.float32),
        pltpu.VMEM((CHUNK, D_OUT), jnp.float32),
        pltpu.VMEM((CHUNK, D_OUT), jnp.float32),
        pltpu.VMEM_SHARED((N, D_OUT), jnp.float32),
        pltpu.SemaphoreType.DMA,
        pltpu.SemaphoreType.DMA,
    ],
)


_BLK = 1000
_NBLK = N // _BLK


def _dotT(a, w):
    return lax.dot_general(a, w, (((1,), (1,)), ((), ())),
                           preferred_element_type=jnp.float32)


def _mlp_body(x_ref, w1_ref, b1_ref, w2_ref, b2_ref, o_ref):
    x = x_ref[0]
    h = _dotT(x, w1_ref[0]) + b1_ref[0]
    o_ref[0] = _dotT(h, w2_ref[0]) + b2_ref[0]


def _mlp(x_list, W1, b1, W2, b2):
    return pl.pallas_call(
        _mlp_body,
        grid=(DURATION, _NBLK),
        in_specs=[
            pl.BlockSpec((1, _BLK, D_IN), lambda t, i: (t, i, 0)),
            pl.BlockSpec((1, D_HID, D_IN), lambda t, i: (t, 0, 0)),
            pl.BlockSpec((1, 1, D_HID), lambda t, i: (t, 0, 0)),
            pl.BlockSpec((1, D_HID, D_HID), lambda t, i: (t, 0, 0)),
            pl.BlockSpec((1, 1, D_HID), lambda t, i: (t, 0, 0)),
        ],
        out_specs=pl.BlockSpec((1, _BLK, D_HID), lambda t, i: (t, i, 0)),
        out_shape=jax.ShapeDtypeStruct((DURATION, N, D_HID), jnp.float32),
    )(x_list, W1, b1.reshape(DURATION, 1, D_HID), W2,
      b2.reshape(DURATION, 1, D_HID))


def _layer_norm(x, g, b, eps=1e-5):
    mu = jnp.mean(x, axis=-1, keepdims=True)
    var = jnp.mean((x - mu) ** 2, axis=-1, keepdims=True)
    return (x - mu) / jnp.sqrt(var + eps) * g + b


def _gru_gates(x_t, h, Wih, Whh, bih, bhh):
    gi = _dotT(x_t, Wih) + bih
    gh = _dotT(h, Whh) + bhh
    r = jax.nn.sigmoid(gi[:, 0:D_OUT] + gh[:, 0:D_OUT])
    z = jax.nn.sigmoid(gi[:, D_OUT:2 * D_OUT] + gh[:, D_OUT:2 * D_OUT])
    n = jnp.tanh(gi[:, 2 * D_OUT:] + r * gh[:, 2 * D_OUT:])
    return (1.0 - z) * n + z * h


def _diff_body(p_ref, wih_ref, whh_ref, bih_ref, bhh_ref, g_ref, b_ref, o_ref):
    p = p_ref[...]
    Wih = wih_ref[...]
    Whh = whh_ref[...]
    bih = bih_ref[...]
    bhh = bhh_ref[...]
    hx = None
    h = jnp.zeros((_BLK, D_OUT), jnp.float32)
    acc = jnp.zeros((_BLK, D_OUT), jnp.float32)
    for c in range(CORE_NUM):
        sp = p[0, c] + p[1, c]
        hx = sp if hx is None else hx + sp
        h = _gru_gates(hx, h, Wih, Whh, bih, bhh)
        acc = acc + h
    o_ref[...] = _layer_norm(acc, g_ref[...], b_ref[...])


def _gru_diff(part, Wih, Whh, bih, bhh, g, b):
    return pl.pallas_call(
        _diff_body,
        grid=(_NBLK,),
        in_specs=[
            pl.BlockSpec((NC, CORE_NUM, _BLK, D_OUT), lambda i: (0, 0, i, 0)),
            pl.BlockSpec((H3, D_OUT), lambda i: (0, 0)),
            pl.BlockSpec((H3, D_OUT), lambda i: (0, 0)),
            pl.BlockSpec((1, H3), lambda i: (0, 0)),
            pl.BlockSpec((1, H3), lambda i: (0, 0)),
            pl.BlockSpec((1, D_OUT), lambda i: (0, 0)),
            pl.BlockSpec((1, D_OUT), lambda i: (0, 0)),
        ],
        out_specs=pl.BlockSpec((_BLK, D_OUT), lambda i: (i, 0)),
        out_shape=jax.ShapeDtypeStruct((N, D_OUT), jnp.float32),
    )(part, Wih, Whh, bih.reshape(1, H3), bhh.reshape(1, H3),
      g.reshape(1, D_OUT), b.reshape(1, D_OUT))


def _rnn_body(hx_ref, wih_ref, whh_ref, bih_ref, bhh_ref, g_ref, b_ref, o_ref):
    hx = hx_ref[...]
    Wih = wih_ref[...]
    Whh = whh_ref[...]
    bih = bih_ref[...]
    bhh = bhh_ref[...]
    g = g_ref[...]
    b = b_ref[...]
    h = jnp.zeros((_BLK, D_OUT), jnp.float32)
    for t in range(DURATION):
        h = _gru_gates(hx[t], h, Wih, Whh, bih, bhh)
        o_ref[t] = _layer_norm(h, g, b)


def _rnn_final(hx, Wih, Whh, bih, bhh, g, b):
    return pl.pallas_call(
        _rnn_body,
        grid=(_NBLK,),
        in_specs=[
            pl.BlockSpec((DURATION, _BLK, D_OUT), lambda i: (0, i, 0)),
            pl.BlockSpec((H3, D_OUT), lambda i: (0, 0)),
            pl.BlockSpec((H3, D_OUT), lambda i: (0, 0)),
            pl.BlockSpec((1, H3), lambda i: (0, 0)),
            pl.BlockSpec((1, H3), lambda i: (0, 0)),
            pl.BlockSpec((1, D_OUT), lambda i: (0, 0)),
            pl.BlockSpec((1, D_OUT), lambda i: (0, 0)),
        ],
        out_specs=pl.BlockSpec((DURATION, _BLK, D_OUT), lambda i: (0, i, 0)),
        out_shape=jax.ShapeDtypeStruct((DURATION, N, D_OUT), jnp.float32),
    )(hx, Wih, Whh, bih.reshape(1, H3), bhh.reshape(1, H3),
      g.reshape(1, D_OUT), b.reshape(1, D_OUT))


def kernel(x_list, edge_index, edge_weight, mlp_W1, mlp_b1, mlp_W2, mlp_b2,
           cd_Wih, cd_Whh, cd_bih, cd_bhh, cd_ln_g, cd_ln_b,
           rnn_Wih, rnn_Whh, rnn_bih, rnn_bhh, norm_g, norm_b):
    x0 = _mlp(x_list, mlp_W1, mlp_b1, mlp_W2, mlp_b2)

    ei32 = edge_index.astype(jnp.int32)
    pad = ((0, 0), (0, 0), (0, 0), (0, EPW_P - EPW))
    src_all = jnp.pad(ei32[:, :, 0, :].reshape(DURATION, CORE_NUM, NW, EPW),
                      pad).reshape(DURATION, CORE_NUM, NW, NCH, CHUNK)
    dst_all = jnp.pad(ei32[:, :, 1, :].reshape(DURATION, CORE_NUM, NW, EPW),
                      pad).reshape(DURATION, CORE_NUM, NW, NCH, CHUNK)
    w_all = jnp.pad(edge_weight.reshape(DURATION, CORE_NUM, NW, EPW),
                    pad).reshape(DURATION, CORE_NUM, NW, NCH, CHUNK)
    zeros625 = jnp.zeros((625, D_OUT), jnp.float32)

    outs = []
    for t in range(DURATION):
        x_cur = x0[t]
        for d in range(DIFF_NUM):
            part = _sc_spmm(x_cur, src_all[t], dst_all[t], w_all[t], zeros625)
            part = part.reshape(NC, CORE_NUM, N, D_OUT)
            x_cur = _gru_diff(part, cd_Wih[t, d], cd_Whh[t, d],
                              cd_bih[t, d], cd_bhh[t, d],
                              cd_ln_g[t, d], cd_ln_b[t, d])
        outs.append(x_cur)

    hx = jnp.stack(outs, axis=0)
    return _rnn_final(hx, rnn_Wih, rnn_Whh, rnn_bih, rnn_bhh, norm_g, norm_b)


# speedup vs baseline: 1.4607x; 1.4607x over previous
import functools

import jax
import jax.numpy as jnp
from jax import lax
from jax.experimental import pallas as pl
from jax.experimental.pallas import tpu as pltpu
from jax.experimental.pallas import tpu_sc as plsc

N = 10000
E = 320000
D_IN = 128
D_HID = 64
D_OUT = 64
DURATION = 3
CORE_NUM = 3
DIFF_NUM = 2
H3 = 3 * D_OUT

NC = 2
NS = 16
NW = NC * NS
EPW = E // NW
CHUNK = 40
NCH = 252
EPW_P = NCH * CHUNK
RING = 6
REFILL = 4


def _sc_spmm_body(x_hbm, src_hbm, dst_hbm, w_hbm, zeros_hbm, out_hbm,
                  src_v, dst_v, w_v, rows, gsems, ssems, slab):
    ci = lax.axis_index("c")
    si = lax.axis_index("s")
    wid = ci * NS + si
    stripe = pl.ds(si * (N // NS), N // NS)

    def scale(rows_v, k):
        kvec = jnp.broadcast_to(k, (16,)).astype(jnp.int32)
        for i in range(CHUNK):
            wspl = plsc.load_gather(w_v, [kvec, jnp.full((16,), i, jnp.int32)])
            for r in range(4):
                rows_v[i, pl.ds(r * 16, 16)] = rows_v[i, pl.ds(r * 16, 16)] * wspl

    def fire_gather(j, m):
        pltpu.async_copy(x_hbm.at[src_v.at[m]], rows[j], gsems[j])

    def wait_gather(j, m):
        pltpu.make_async_copy(x_hbm.at[src_v.at[m]], rows[j], gsems[j]).wait()

    def fire_scatter(j, m):
        pltpu.async_copy(rows[j], slab.at[dst_v.at[m]], ssems[j], add=True)

    def wait_scatter(j, m):
        pltpu.make_async_copy(rows[j], slab.at[dst_v.at[m]], ssems[j]).wait()

    for cc in range(CORE_NUM):
        pltpu.sync_copy(zeros_hbm, slab.at[stripe])
        pltpu.sync_copy(src_hbm.at[cc, wid], src_v)
        pltpu.sync_copy(dst_hbm.at[cc, wid], dst_v)
        pltpu.sync_copy(w_hbm.at[cc, wid], w_v)
        plsc.subcore_barrier()

        for j in range(REFILL):
            fire_gather(j, j)

        def ring_body(k6, carry):
            k = RING * k6
            for j in range(RING):
                m = k + j
                wait_gather(j, m)
                scale(rows[j], m)
                fire_scatter(j, m)
                jr = (j + REFILL) % RING
                mn = m + REFILL

                @pl.when(mn < NCH)
                def _():
                    @pl.when(m >= RING - REFILL)
                    def _():
                        wait_scatter(jr, m - (RING - REFILL))
                    fire_gather(jr, mn)
            return carry

        lax.fori_loop(0, NCH // RING, ring_body, 0)

        for j in range(RING):
            wait_scatter(j, NCH - RING + j)

        plsc.subcore_barrier()

        pltpu.sync_copy(slab.at[stripe], out_hbm.at[ci, cc, si])
        plsc.subcore_barrier()


_sc_spmm = pl.kernel(
    _sc_spmm_body,
    out_type=jax.ShapeDtypeStruct((NC, CORE_NUM, NS, N // NS, D_OUT),
                                  jnp.float32),
    mesh=plsc.VectorSubcoreMesh(core_axis_name="c", subcore_axis_name="s",
                                num_cores=NC, num_subcores=NS),
    compiler_params=pltpu.CompilerParams(needs_layout_passes=False,
                                         use_tc_tiling_on_sc=False),
    scratch_types=[
        pltpu.VMEM((NCH, CHUNK), jnp.int32),
        pltpu.VMEM((NCH, CHUNK), jnp.int32),
        pltpu.VMEM((NCH, CHUNK), jnp.float32),
        [pltpu.VMEM((CHUNK, D_OUT), jnp.float32) for _ in range(RING)],
        [pltpu.SemaphoreType.DMA for _ in range(RING)],
        [pltpu.SemaphoreType.DMA for _ in range(RING)],
        pltpu.VMEM_SHARED((N, D_OUT), jnp.float32),
    ],
)


_BLK = 1000
_NBLK = N // _BLK


def _dotT(a, w):
    return lax.dot_general(a, w, (((1,), (1,)), ((), ())),
                           preferred_element_type=jnp.float32)


def _mlp_body(x_ref, w1_ref, b1_ref, w2_ref, b2_ref, o_ref):
    x = x_ref[0]
    h = _dotT(x, w1_ref[0]) + b1_ref[0]
    o_ref[0] = _dotT(h, w2_ref[0]) + b2_ref[0]


def _mlp(x_list, W1, b1, W2, b2):
    return pl.pallas_call(
        _mlp_body,
        grid=(DURATION, _NBLK),
        in_specs=[
            pl.BlockSpec((1, _BLK, D_IN), lambda t, i: (t, i, 0)),
            pl.BlockSpec((1, D_HID, D_IN), lambda t, i: (t, 0, 0)),
            pl.BlockSpec((1, 1, D_HID), lambda t, i: (t, 0, 0)),
            pl.BlockSpec((1, D_HID, D_HID), lambda t, i: (t, 0, 0)),
            pl.BlockSpec((1, 1, D_HID), lambda t, i: (t, 0, 0)),
        ],
        out_specs=pl.BlockSpec((1, _BLK, D_HID), lambda t, i: (t, i, 0)),
        out_shape=jax.ShapeDtypeStruct((DURATION, N, D_HID), jnp.float32),
    )(x_list, W1, b1.reshape(DURATION, 1, D_HID), W2,
      b2.reshape(DURATION, 1, D_HID))


def _layer_norm(x, g, b, eps=1e-5):
    mu = jnp.mean(x, axis=-1, keepdims=True)
    var = jnp.mean((x - mu) ** 2, axis=-1, keepdims=True)
    return (x - mu) / jnp.sqrt(var + eps) * g + b


def _gru_gates(x_t, h, Wih, Whh, bih, bhh):
    gi = _dotT(x_t, Wih) + bih
    gh = _dotT(h, Whh) + bhh
    r = jax.nn.sigmoid(gi[:, 0:D_OUT] + gh[:, 0:D_OUT])
    z = jax.nn.sigmoid(gi[:, D_OUT:2 * D_OUT] + gh[:, D_OUT:2 * D_OUT])
    n = jnp.tanh(gi[:, 2 * D_OUT:] + r * gh[:, 2 * D_OUT:])
    return (1.0 - z) * n + z * h


def _diff_body(p_ref, wih_ref, whh_ref, bih_ref, bhh_ref, g_ref, b_ref, o_ref):
    p = p_ref[...]
    Wih = wih_ref[...]
    Whh = whh_ref[...]
    bih = bih_ref[...]
    bhh = bhh_ref[...]
    hx = None
    h = jnp.zeros((_BLK, D_OUT), jnp.float32)
    acc = jnp.zeros((_BLK, D_OUT), jnp.float32)
    for c in range(CORE_NUM):
        sp = p[0, c] + p[1, c]
        hx = sp if hx is None else hx + sp
        h = _gru_gates(hx, h, Wih, Whh, bih, bhh)
        acc = acc + h
    o_ref[...] = _layer_norm(acc, g_ref[...], b_ref[...])


def _gru_diff(part, Wih, Whh, bih, bhh, g, b):
    return pl.pallas_call(
        _diff_body,
        grid=(_NBLK,),
        in_specs=[
            pl.BlockSpec((NC, CORE_NUM, _BLK, D_OUT), lambda i: (0, 0, i, 0)),
            pl.BlockSpec((H3, D_OUT), lambda i: (0, 0)),
            pl.BlockSpec((H3, D_OUT), lambda i: (0, 0)),
            pl.BlockSpec((1, H3), lambda i: (0, 0)),
            pl.BlockSpec((1, H3), lambda i: (0, 0)),
            pl.BlockSpec((1, D_OUT), lambda i: (0, 0)),
            pl.BlockSpec((1, D_OUT), lambda i: (0, 0)),
        ],
        out_specs=pl.BlockSpec((_BLK, D_OUT), lambda i: (i, 0)),
        out_shape=jax.ShapeDtypeStruct((N, D_OUT), jnp.float32),
    )(part, Wih, Whh, bih.reshape(1, H3), bhh.reshape(1, H3),
      g.reshape(1, D_OUT), b.reshape(1, D_OUT))


def _rnn_body(hx_ref, wih_ref, whh_ref, bih_ref, bhh_ref, g_ref, b_ref, o_ref):
    hx = hx_ref[...]
    Wih = wih_ref[...]
    Whh = whh_ref[...]
    bih = bih_ref[...]
    bhh = bhh_ref[...]
    g = g_ref[...]
    b = b_ref[...]
    h = jnp.zeros((_BLK, D_OUT), jnp.float32)
    for t in range(DURATION):
        h = _gru_gates(hx[t], h, Wih, Whh, bih, bhh)
        o_ref[t] = _layer_norm(h, g, b)


def _rnn_final(hx, Wih, Whh, bih, bhh, g, b):
    return pl.pallas_call(
        _rnn_body,
        grid=(_NBLK,),
        in_specs=[
            pl.BlockSpec((DURATION, _BLK, D_OUT), lambda i: (0, i, 0)),
            pl.BlockSpec((H3, D_OUT), lambda i: (0, 0)),
            pl.BlockSpec((H3, D_OUT), lambda i: (0, 0)),
            pl.BlockSpec((1, H3), lambda i: (0, 0)),
            pl.BlockSpec((1, H3), lambda i: (0, 0)),
            pl.BlockSpec((1, D_OUT), lambda i: (0, 0)),
            pl.BlockSpec((1, D_OUT), lambda i: (0, 0)),
        ],
        out_specs=pl.BlockSpec((DURATION, _BLK, D_OUT), lambda i: (0, i, 0)),
        out_shape=jax.ShapeDtypeStruct((DURATION, N, D_OUT), jnp.float32),
    )(hx, Wih, Whh, bih.reshape(1, H3), bhh.reshape(1, H3),
      g.reshape(1, D_OUT), b.reshape(1, D_OUT))


def kernel(x_list, edge_index, edge_weight, mlp_W1, mlp_b1, mlp_W2, mlp_b2,
           cd_Wih, cd_Whh, cd_bih, cd_bhh, cd_ln_g, cd_ln_b,
           rnn_Wih, rnn_Whh, rnn_bih, rnn_bhh, norm_g, norm_b):
    x0 = _mlp(x_list, mlp_W1, mlp_b1, mlp_W2, mlp_b2)

    ei32 = edge_index.astype(jnp.int32)
    pad = ((0, 0), (0, 0), (0, 0), (0, EPW_P - EPW))
    src_all = jnp.pad(ei32[:, :, 0, :].reshape(DURATION, CORE_NUM, NW, EPW),
                      pad).reshape(DURATION, CORE_NUM, NW, NCH, CHUNK)
    dst_all = jnp.pad(ei32[:, :, 1, :].reshape(DURATION, CORE_NUM, NW, EPW),
                      pad).reshape(DURATION, CORE_NUM, NW, NCH, CHUNK)
    w_all = jnp.pad(edge_weight.reshape(DURATION, CORE_NUM, NW, EPW),
                    pad).reshape(DURATION, CORE_NUM, NW, NCH, CHUNK)
    zeros625 = jnp.zeros((625, D_OUT), jnp.float32)

    outs = []
    for t in range(DURATION):
        x_cur = x0[t]
        for d in range(DIFF_NUM):
            part = _sc_spmm(x_cur, src_all[t], dst_all[t], w_all[t], zeros625)
            part = part.reshape(NC, CORE_NUM, N, D_OUT)
            x_cur = _gru_diff(part, cd_Wih[t, d], cd_Whh[t, d],
                              cd_bih[t, d], cd_bhh[t, d],
                              cd_ln_g[t, d], cd_ln_b[t, d])
        outs.append(x_cur)

    hx = jnp.stack(outs, axis=0)
    return _rnn_final(hx, rnn_Wih, rnn_Whh, rnn_bih, rnn_bhh, norm_g, norm_b)


# speedup vs baseline: 1.6126x; 1.1040x over previous
import functools

import jax
import jax.numpy as jnp
from jax import lax
from jax.experimental import pallas as pl
from jax.experimental.pallas import tpu as pltpu
from jax.experimental.pallas import tpu_sc as plsc

N = 10000
E = 320000
D_IN = 128
D_HID = 64
D_OUT = 64
DURATION = 3
CORE_NUM = 3
DIFF_NUM = 2
H3 = 3 * D_OUT

NC = 2
NS = 16
NW = NC * NS
EPW = E // NW
CHUNK = 40
NCH = 252
EPW_P = NCH * CHUNK
RING = 6
REFILL = 4


def _sc_spmm_body(x_hbm, src_hbm, dst_hbm, w_hbm, zeros_hbm, out_hbm,
                  src_v, dst_v, w_v, rows, gsems, ssems, slab, xs):
    ci = lax.axis_index("c")
    si = lax.axis_index("s")
    wid = ci * NS + si
    stripe = pl.ds(si * (N // NS), N // NS)

    pltpu.sync_copy(x_hbm.at[stripe], xs.at[stripe])

    def scale(rows_v, k):
        kvec = jnp.broadcast_to(k, (16,)).astype(jnp.int32)
        for i in range(CHUNK):
            wspl = plsc.load_gather(w_v, [kvec, jnp.full((16,), i, jnp.int32)])
            for r in range(4):
                rows_v[i, pl.ds(r * 16, 16)] = rows_v[i, pl.ds(r * 16, 16)] * wspl

    def fire_gather(j, m):
        pltpu.async_copy(xs.at[src_v.at[m]], rows[j], gsems[j])

    def wait_gather(j, m):
        pltpu.make_async_copy(xs.at[src_v.at[m]], rows[j], gsems[j]).wait()

    def fire_scatter(j, m):
        pltpu.async_copy(rows[j], slab.at[dst_v.at[m]], ssems[j], add=True)

    def wait_scatter(j, m):
        pltpu.make_async_copy(rows[j], slab.at[dst_v.at[m]], ssems[j]).wait()

    for cc in range(CORE_NUM):
        pltpu.sync_copy(zeros_hbm, slab.at[stripe])
        pltpu.sync_copy(src_hbm.at[cc, wid], src_v)
        pltpu.sync_copy(dst_hbm.at[cc, wid], dst_v)
        pltpu.sync_copy(w_hbm.at[cc, wid], w_v)
        plsc.subcore_barrier()

        for j in range(REFILL):
            fire_gather(j, j)

        def ring_body(k6, carry):
            k = RING * k6
            for j in range(RING):
                m = k + j
                wait_gather(j, m)
                scale(rows[j], m)
                fire_scatter(j, m)
                jr = (j + REFILL) % RING
                mn = m + REFILL

                @pl.when(mn < NCH)
                def _():
                    @pl.when(m >= RING - REFILL)
                    def _():
                        wait_scatter(jr, m - (RING - REFILL))
                    fire_gather(jr, mn)
            return carry

        lax.fori_loop(0, NCH // RING, ring_body, 0)

        for j in range(RING):
            wait_scatter(j, NCH - RING + j)

        plsc.subcore_barrier()

        pltpu.sync_copy(slab.at[stripe], out_hbm.at[ci, cc, si])
        plsc.subcore_barrier()


_sc_spmm = pl.kernel(
    _sc_spmm_body,
    out_type=jax.ShapeDtypeStruct((NC, CORE_NUM, NS, N // NS, D_OUT),
                                  jnp.float32),
    mesh=plsc.VectorSubcoreMesh(core_axis_name="c", subcore_axis_name="s",
                                num_cores=NC, num_subcores=NS),
    compiler_params=pltpu.CompilerParams(needs_layout_passes=False,
                                         use_tc_tiling_on_sc=False),
    scratch_types=[
        pltpu.VMEM((NCH, CHUNK), jnp.int32),
        pltpu.VMEM((NCH, CHUNK), jnp.int32),
        pltpu.VMEM((NCH, CHUNK), jnp.float32),
        [pltpu.VMEM((CHUNK, D_OUT), jnp.float32) for _ in range(RING)],
        [pltpu.SemaphoreType.DMA for _ in range(RING)],
        [pltpu.SemaphoreType.DMA for _ in range(RING)],
        pltpu.VMEM_SHARED((N, D_OUT), jnp.float32),
        pltpu.VMEM_SHARED((N, D_OUT), jnp.float32),
    ],
)


_BLK = 1000
_NBLK = N // _BLK


def _dotT(a, w):
    return lax.dot_general(a, w, (((1,), (1,)), ((), ())),
                           preferred_element_type=jnp.float32)


def _mlp_body(x_ref, w1_ref, b1_ref, w2_ref, b2_ref, o_ref):
    x = x_ref[0]
    h = _dotT(x, w1_ref[0]) + b1_ref[0]
    o_ref[0] = _dotT(h, w2_ref[0]) + b2_ref[0]


def _mlp(x_list, W1, b1, W2, b2):
    return pl.pallas_call(
        _mlp_body,
        grid=(DURATION, _NBLK),
        in_specs=[
            pl.BlockSpec((1, _BLK, D_IN), lambda t, i: (t, i, 0)),
            pl.BlockSpec((1, D_HID, D_IN), lambda t, i: (t, 0, 0)),
            pl.BlockSpec((1, 1, D_HID), lambda t, i: (t, 0, 0)),
            pl.BlockSpec((1, D_HID, D_HID), lambda t, i: (t, 0, 0)),
            pl.BlockSpec((1, 1, D_HID), lambda t, i: (t, 0, 0)),
        ],
        out_specs=pl.BlockSpec((1, _BLK, D_HID), lambda t, i: (t, i, 0)),
        out_shape=jax.ShapeDtypeStruct((DURATION, N, D_HID), jnp.float32),
    )(x_list, W1, b1.reshape(DURATION, 1, D_HID), W2,
      b2.reshape(DURATION, 1, D_HID))


def _layer_norm(x, g, b, eps=1e-5):
    mu = jnp.mean(x, axis=-1, keepdims=True)
    var = jnp.mean((x - mu) ** 2, axis=-1, keepdims=True)
    return (x - mu) / jnp.sqrt(var + eps) * g + b


def _gru_gates(x_t, h, Wih, Whh, bih, bhh):
    gi = _dotT(x_t, Wih) + bih
    gh = _dotT(h, Whh) + bhh
    r = jax.nn.sigmoid(gi[:, 0:D_OUT] + gh[:, 0:D_OUT])
    z = jax.nn.sigmoid(gi[:, D_OUT:2 * D_OUT] + gh[:, D_OUT:2 * D_OUT])
    n = jnp.tanh(gi[:, 2 * D_OUT:] + r * gh[:, 2 * D_OUT:])
    return (1.0 - z) * n + z * h


def _diff_body(p_ref, wih_ref, whh_ref, bih_ref, bhh_ref, g_ref, b_ref, o_ref):
    p = p_ref[...]
    Wih = wih_ref[...]
    Whh = whh_ref[...]
    bih = bih_ref[...]
    bhh = bhh_ref[...]
    hx = None
    h = jnp.zeros((_BLK, D_OUT), jnp.float32)
    acc = jnp.zeros((_BLK, D_OUT), jnp.float32)
    for c in range(CORE_NUM):
        sp = p[0, c] + p[1, c]
        hx = sp if hx is None else hx + sp
        h = _gru_gates(hx, h, Wih, Whh, bih, bhh)
        acc = acc + h
    o_ref[...] = _layer_norm(acc, g_ref[...], b_ref[...])


def _gru_diff(part, Wih, Whh, bih, bhh, g, b):
    return pl.pallas_call(
        _diff_body,
        grid=(_NBLK,),
        in_specs=[
            pl.BlockSpec((NC, CORE_NUM, _BLK, D_OUT), lambda i: (0, 0, i, 0)),
            pl.BlockSpec((H3, D_OUT), lambda i: (0, 0)),
            pl.BlockSpec((H3, D_OUT), lambda i: (0, 0)),
            pl.BlockSpec((1, H3), lambda i: (0, 0)),
            pl.BlockSpec((1, H3), lambda i: (0, 0)),
            pl.BlockSpec((1, D_OUT), lambda i: (0, 0)),
            pl.BlockSpec((1, D_OUT), lambda i: (0, 0)),
        ],
        out_specs=pl.BlockSpec((_BLK, D_OUT), lambda i: (i, 0)),
        out_shape=jax.ShapeDtypeStruct((N, D_OUT), jnp.float32),
    )(part, Wih, Whh, bih.reshape(1, H3), bhh.reshape(1, H3),
      g.reshape(1, D_OUT), b.reshape(1, D_OUT))


def _rnn_body(hx_ref, wih_ref, whh_ref, bih_ref, bhh_ref, g_ref, b_ref, o_ref):
    hx = hx_ref[...]
    Wih = wih_ref[...]
    Whh = whh_ref[...]
    bih = bih_ref[...]
    bhh = bhh_ref[...]
    g = g_ref[...]
    b = b_ref[...]
    h = jnp.zeros((_BLK, D_OUT), jnp.float32)
    for t in range(DURATION):
        h = _gru_gates(hx[t], h, Wih, Whh, bih, bhh)
        o_ref[t] = _layer_norm(h, g, b)


def _rnn_final(hx, Wih, Whh, bih, bhh, g, b):
    return pl.pallas_call(
        _rnn_body,
        grid=(_NBLK,),
        in_specs=[
            pl.BlockSpec((DURATION, _BLK, D_OUT), lambda i: (0, i, 0)),
            pl.BlockSpec((H3, D_OUT), lambda i: (0, 0)),
            pl.BlockSpec((H3, D_OUT), lambda i: (0, 0)),
            pl.BlockSpec((1, H3), lambda i: (0, 0)),
            pl.BlockSpec((1, H3), lambda i: (0, 0)),
            pl.BlockSpec((1, D_OUT), lambda i: (0, 0)),
            pl.BlockSpec((1, D_OUT), lambda i: (0, 0)),
        ],
        out_specs=pl.BlockSpec((DURATION, _BLK, D_OUT), lambda i: (0, i, 0)),
        out_shape=jax.ShapeDtypeStruct((DURATION, N, D_OUT), jnp.float32),
    )(hx, Wih, Whh, bih.reshape(1, H3), bhh.reshape(1, H3),
      g.reshape(1, D_OUT), b.reshape(1, D_OUT))


def kernel(x_list, edge_index, edge_weight, mlp_W1, mlp_b1, mlp_W2, mlp_b2,
           cd_Wih, cd_Whh, cd_bih, cd_bhh, cd_ln_g, cd_ln_b,
           rnn_Wih, rnn_Whh, rnn_bih, rnn_bhh, norm_g, norm_b):
    x0 = _mlp(x_list, mlp_W1, mlp_b1, mlp_W2, mlp_b2)

    ei32 = edge_index.astype(jnp.int32)
    pad = ((0, 0), (0, 0), (0, 0), (0, EPW_P - EPW))
    src_all = jnp.pad(ei32[:, :, 0, :].reshape(DURATION, CORE_NUM, NW, EPW),
                      pad).reshape(DURATION, CORE_NUM, NW, NCH, CHUNK)
    dst_all = jnp.pad(ei32[:, :, 1, :].reshape(DURATION, CORE_NUM, NW, EPW),
                      pad).reshape(DURATION, CORE_NUM, NW, NCH, CHUNK)
    w_all = jnp.pad(edge_weight.reshape(DURATION, CORE_NUM, NW, EPW),
                    pad).reshape(DURATION, CORE_NUM, NW, NCH, CHUNK)
    zeros625 = jnp.zeros((625, D_OUT), jnp.float32)

    outs = []
    for t in range(DURATION):
        x_cur = x0[t]
        for d in range(DIFF_NUM):
            part = _sc_spmm(x_cur, src_all[t], dst_all[t], w_all[t], zeros625)
            part = part.reshape(NC, CORE_NUM, N, D_OUT)
            x_cur = _gru_diff(part, cd_Wih[t, d], cd_Whh[t, d],
                              cd_bih[t, d], cd_bhh[t, d],
                              cd_ln_g[t, d], cd_ln_b[t, d])
        outs.append(x_cur)

    hx = jnp.stack(outs, axis=0)
    return _rnn_final(hx, rnn_Wih, rnn_Whh, rnn_bih, rnn_bhh, norm_g, norm_b)


# speedup vs baseline: 1.7044x; 1.0569x over previous
import functools

import jax
import jax.numpy as jnp
from jax import lax
from jax.experimental import pallas as pl
from jax.experimental.pallas import tpu as pltpu
from jax.experimental.pallas import tpu_sc as plsc

N = 10000
E = 320000
D_IN = 128
D_HID = 64
D_OUT = 64
DURATION = 3
CORE_NUM = 3
DIFF_NUM = 2
H3 = 3 * D_OUT

NC = 2
NS = 16
NW = NC * NS
EPW = E // NW
CHUNK = 40
NCH = 252
EPW_P = NCH * CHUNK
RING = 4
REFILL = 3


def _sc_spmm_body(x_hbm, src_hbm, dst_hbm, w_hbm, zeros_hbm, out_hbm,
                  src_v, dst_v, w_v, rows, scaled, gsems, ssems, slab, xs):
    ci = lax.axis_index("c")
    si = lax.axis_index("s")
    wid = ci * NS + si
    stripe = pl.ds(si * (N // NS), N // NS)

    pltpu.sync_copy(x_hbm.at[stripe], xs.at[stripe])

    def scale(rows_v, out_v, k):
        kvec = jnp.broadcast_to(k, (16,)).astype(jnp.int32)
        for i in range(CHUNK):
            wspl = plsc.load_gather(w_v, [kvec, jnp.full((16,), i, jnp.int32)])
            for r in range(4):
                out_v[i, pl.ds(r * 16, 16)] = rows_v[i, pl.ds(r * 16, 16)] * wspl

    def fire_gather(j, m):
        pltpu.async_copy(xs.at[src_v.at[m]], rows[j], gsems[j])

    def wait_gather(j, m):
        pltpu.make_async_copy(xs.at[src_v.at[m]], rows[j], gsems[j]).wait()

    def fire_scatter(j, m):
        pltpu.async_copy(scaled[j], slab.at[dst_v.at[m]], ssems[j], add=True)

    def wait_scatter(j, m):
        pltpu.make_async_copy(scaled[j], slab.at[dst_v.at[m]], ssems[j]).wait()

    for cc in range(CORE_NUM):
        pltpu.sync_copy(zeros_hbm, slab.at[stripe])
        pltpu.sync_copy(src_hbm.at[cc, wid], src_v)
        pltpu.sync_copy(dst_hbm.at[cc, wid], dst_v)
        pltpu.sync_copy(w_hbm.at[cc, wid], w_v)
        plsc.subcore_barrier()

        for j in range(REFILL):
            fire_gather(j, j)

        def ring_body(k4, carry):
            k = RING * k4
            for j in range(RING):
                m = k + j
                wait_gather(j, m)

                @pl.when(m >= RING)
                def _():
                    wait_scatter(j, m - RING)

                scale(rows[j], scaled[j], m)
                fire_scatter(j, m)
                mn = m + REFILL

                @pl.when(mn < NCH)
                def _():
                    fire_gather((j + REFILL) % RING, mn)
            return carry

        lax.fori_loop(0, NCH // RING, ring_body, 0)

        for j in range(RING):
            wait_scatter(j, NCH - RING + j)

        plsc.subcore_barrier()

        pltpu.sync_copy(slab.at[stripe], out_hbm.at[ci, cc, si])
        plsc.subcore_barrier()


_sc_spmm = pl.kernel(
    _sc_spmm_body,
    out_type=jax.ShapeDtypeStruct((NC, CORE_NUM, NS, N // NS, D_OUT),
                                  jnp.float32),
    mesh=plsc.VectorSubcoreMesh(core_axis_name="c", subcore_axis_name="s",
                                num_cores=NC, num_subcores=NS),
    compiler_params=pltpu.CompilerParams(needs_layout_passes=False,
                                         use_tc_tiling_on_sc=False),
    scratch_types=[
        pltpu.VMEM((NCH, CHUNK), jnp.int32),
        pltpu.VMEM((NCH, CHUNK), jnp.int32),
        pltpu.VMEM((NCH, CHUNK), jnp.float32),
        [pltpu.VMEM((CHUNK, D_OUT), jnp.float32) for _ in range(RING)],
        [pltpu.VMEM((CHUNK, D_OUT), jnp.float32) for _ in range(RING)],
        [pltpu.SemaphoreType.DMA for _ in range(RING)],
        [pltpu.SemaphoreType.DMA for _ in range(RING)],
        pltpu.VMEM_SHARED((N, D_OUT), jnp.float32),
        pltpu.VMEM_SHARED((N, D_OUT), jnp.float32),
    ],
)


_BLK = 1000
_NBLK = N // _BLK


def _dotT(a, w):
    return lax.dot_general(a, w, (((1,), (1,)), ((), ())),
                           preferred_element_type=jnp.float32)


def _mlp_body(x_ref, w1_ref, b1_ref, w2_ref, b2_ref, o_ref):
    x = x_ref[0]
    h = _dotT(x, w1_ref[0]) + b1_ref[0]
    o_ref[0] = _dotT(h, w2_ref[0]) + b2_ref[0]


def _mlp(x_list, W1, b1, W2, b2):
    return pl.pallas_call(
        _mlp_body,
        grid=(DURATION, _NBLK),
        in_specs=[
            pl.BlockSpec((1, _BLK, D_IN), lambda t, i: (t, i, 0)),
            pl.BlockSpec((1, D_HID, D_IN), lambda t, i: (t, 0, 0)),
            pl.BlockSpec((1, 1, D_HID), lambda t, i: (t, 0, 0)),
            pl.BlockSpec((1, D_HID, D_HID), lambda t, i: (t, 0, 0)),
            pl.BlockSpec((1, 1, D_HID), lambda t, i: (t, 0, 0)),
        ],
        out_specs=pl.BlockSpec((1, _BLK, D_HID), lambda t, i: (t, i, 0)),
        out_shape=jax.ShapeDtypeStruct((DURATION, N, D_HID), jnp.float32),
    )(x_list, W1, b1.reshape(DURATION, 1, D_HID), W2,
      b2.reshape(DURATION, 1, D_HID))


def _layer_norm(x, g, b, eps=1e-5):
    mu = jnp.mean(x, axis=-1, keepdims=True)
    var = jnp.mean((x - mu) ** 2, axis=-1, keepdims=True)
    return (x - mu) / jnp.sqrt(var + eps) * g + b


def _gru_gates(x_t, h, Wih, Whh, bih, bhh):
    gi = _dotT(x_t, Wih) + bih
    gh = _dotT(h, Whh) + bhh
    r = jax.nn.sigmoid(gi[:, 0:D_OUT] + gh[:, 0:D_OUT])
    z = jax.nn.sigmoid(gi[:, D_OUT:2 * D_OUT] + gh[:, D_OUT:2 * D_OUT])
    n = jnp.tanh(gi[:, 2 * D_OUT:] + r * gh[:, 2 * D_OUT:])
    return (1.0 - z) * n + z * h


def _diff_body(p_ref, wih_ref, whh_ref, bih_ref, bhh_ref, g_ref, b_ref, o_ref):
    p = p_ref[...]
    Wih = wih_ref[...]
    Whh = whh_ref[...]
    bih = bih_ref[...]
    bhh = bhh_ref[...]
    hx = None
    h = jnp.zeros((_BLK, D_OUT), jnp.float32)
    acc = jnp.zeros((_BLK, D_OUT), jnp.float32)
    for c in range(CORE_NUM):
        sp = p[0, c] + p[1, c]
        hx = sp if hx is None else hx + sp
        h = _gru_gates(hx, h, Wih, Whh, bih, bhh)
        acc = acc + h
    o_ref[...] = _layer_norm(acc, g_ref[...], b_ref[...])


def _gru_diff(part, Wih, Whh, bih, bhh, g, b):
    return pl.pallas_call(
        _diff_body,
        grid=(_NBLK,),
        in_specs=[
            pl.BlockSpec((NC, CORE_NUM, _BLK, D_OUT), lambda i: (0, 0, i, 0)),
            pl.BlockSpec((H3, D_OUT), lambda i: (0, 0)),
            pl.BlockSpec((H3, D_OUT), lambda i: (0, 0)),
            pl.BlockSpec((1, H3), lambda i: (0, 0)),
            pl.BlockSpec((1, H3), lambda i: (0, 0)),
            pl.BlockSpec((1, D_OUT), lambda i: (0, 0)),
            pl.BlockSpec((1, D_OUT), lambda i: (0, 0)),
        ],
        out_specs=pl.BlockSpec((_BLK, D_OUT), lambda i: (i, 0)),
        out_shape=jax.ShapeDtypeStruct((N, D_OUT), jnp.float32),
    )(part, Wih, Whh, bih.reshape(1, H3), bhh.reshape(1, H3),
      g.reshape(1, D_OUT), b.reshape(1, D_OUT))


def _rnn_body(hx_ref, wih_ref, whh_ref, bih_ref, bhh_ref, g_ref, b_ref, o_ref):
    hx = hx_ref[...]
    Wih = wih_ref[...]
    Whh = whh_ref[...]
    bih = bih_ref[...]
    bhh = bhh_ref[...]
    g = g_ref[...]
    b = b_ref[...]
    h = jnp.zeros((_BLK, D_OUT), jnp.float32)
    for t in range(DURATION):
        h = _gru_gates(hx[t], h, Wih, Whh, bih, bhh)
        o_ref[t] = _layer_norm(h, g, b)


def _rnn_final(hx, Wih, Whh, bih, bhh, g, b):
    return pl.pallas_call(
        _rnn_body,
        grid=(_NBLK,),
        in_specs=[
            pl.BlockSpec((DURATION, _BLK, D_OUT), lambda i: (0, i, 0)),
            pl.BlockSpec((H3, D_OUT), lambda i: (0, 0)),
            pl.BlockSpec((H3, D_OUT), lambda i: (0, 0)),
            pl.BlockSpec((1, H3), lambda i: (0, 0)),
            pl.BlockSpec((1, H3), lambda i: (0, 0)),
            pl.BlockSpec((1, D_OUT), lambda i: (0, 0)),
            pl.BlockSpec((1, D_OUT), lambda i: (0, 0)),
        ],
        out_specs=pl.BlockSpec((DURATION, _BLK, D_OUT), lambda i: (0, i, 0)),
        out_shape=jax.ShapeDtypeStruct((DURATION, N, D_OUT), jnp.float32),
    )(hx, Wih, Whh, bih.reshape(1, H3), bhh.reshape(1, H3),
      g.reshape(1, D_OUT), b.reshape(1, D_OUT))


def kernel(x_list, edge_index, edge_weight, mlp_W1, mlp_b1, mlp_W2, mlp_b2,
           cd_Wih, cd_Whh, cd_bih, cd_bhh, cd_ln_g, cd_ln_b,
           rnn_Wih, rnn_Whh, rnn_bih, rnn_bhh, norm_g, norm_b):
    x0 = _mlp(x_list, mlp_W1, mlp_b1, mlp_W2, mlp_b2)

    ei32 = edge_index.astype(jnp.int32)
    pad = ((0, 0), (0, 0), (0, 0), (0, EPW_P - EPW))
    src_all = jnp.pad(ei32[:, :, 0, :].reshape(DURATION, CORE_NUM, NW, EPW),
                      pad).reshape(DURATION, CORE_NUM, NW, NCH, CHUNK)
    dst_all = jnp.pad(ei32[:, :, 1, :].reshape(DURATION, CORE_NUM, NW, EPW),
                      pad).reshape(DURATION, CORE_NUM, NW, NCH, CHUNK)
    w_all = jnp.pad(edge_weight.reshape(DURATION, CORE_NUM, NW, EPW),
                    pad).reshape(DURATION, CORE_NUM, NW, NCH, CHUNK)
    zeros625 = jnp.zeros((625, D_OUT), jnp.float32)

    outs = []
    for t in range(DURATION):
        x_cur = x0[t]
        for d in range(DIFF_NUM):
            part = _sc_spmm(x_cur, src_all[t], dst_all[t], w_all[t], zeros625)
            part = part.reshape(NC, CORE_NUM, N, D_OUT)
            x_cur = _gru_diff(part, cd_Wih[t, d], cd_Whh[t, d],
                              cd_bih[t, d], cd_bhh[t, d],
                              cd_ln_g[t, d], cd_ln_b[t, d])
        outs.append(x_cur)

    hx = jnp.stack(outs, axis=0)
    return _rnn_final(hx, rnn_Wih, rnn_Whh, rnn_bih, rnn_bhh, norm_g, norm_b)


# speedup vs baseline: 1.7432x; 1.0228x over previous
import functools

import jax
import jax.numpy as jnp
from jax import lax
from jax.experimental import pallas as pl
from jax.experimental.pallas import tpu as pltpu
from jax.experimental.pallas import tpu_sc as plsc

N = 10000
E = 320000
D_IN = 128
D_HID = 64
D_OUT = 64
DURATION = 3
CORE_NUM = 3
DIFF_NUM = 2
H3 = 3 * D_OUT

NC = 2
NS = 16
NW = NC * NS
EPW = E // NW
CHUNK = 40
NCH = 252
EPW_P = NCH * CHUNK
RING = 4
REFILL = 3


def _sc_spmm_body(x_hbm, src_hbm, dst_hbm, w_hbm, zeros_hbm, out_hbm,
                  src_v, dst_v, w_v, rows, scaled, gsems, ssems, slab, xs):
    ci = lax.axis_index("c")
    si = lax.axis_index("s")
    wid = ci * NS + si
    stripe = pl.ds(si * (N // NS), N // NS)

    pltpu.sync_copy(x_hbm.at[stripe], xs.at[stripe])

    def scale(rows_v, out_v, k):
        kvec = jnp.broadcast_to(k, (16,)).astype(jnp.int32)
        for i in range(CHUNK):
            wspl = plsc.load_gather(w_v, [kvec, jnp.full((16,), i, jnp.int32)])
            for r in range(4):
                out_v[i, pl.ds(r * 16, 16)] = rows_v[i, pl.ds(r * 16, 16)] * wspl

    def fire_gather(j, m):
        pltpu.async_copy(xs.at[src_v.at[m]], rows[j], gsems[j])

    def wait_gather(j, m):
        pltpu.make_async_copy(xs.at[src_v.at[m]], rows[j], gsems[j]).wait()

    def fire_scatter(j, m):
        pltpu.async_copy(scaled[j], slab.at[dst_v.at[m]], ssems[j], add=True)

    def wait_scatter(j, m):
        pltpu.make_async_copy(scaled[j], slab.at[dst_v.at[m]], ssems[j]).wait()

    def core_phase(cc, carry):
        pltpu.sync_copy(zeros_hbm, slab.at[stripe])
        pltpu.sync_copy(src_hbm.at[cc, wid], src_v)
        pltpu.sync_copy(dst_hbm.at[cc, wid], dst_v)
        pltpu.sync_copy(w_hbm.at[cc, wid], w_v)
        plsc.subcore_barrier()

        for j in range(REFILL):
            fire_gather(j, j)

        def ring_body(k4, carry):
            k = RING * k4
            for j in range(RING):
                m = k + j
                wait_gather(j, m)

                @pl.when(m >= RING)
                def _():
                    wait_scatter(j, m - RING)

                scale(rows[j], scaled[j], m)
                fire_scatter(j, m)
                mn = m + REFILL

                @pl.when(mn < NCH)
                def _():
                    fire_gather((j + REFILL) % RING, mn)
            return carry

        lax.fori_loop(0, NCH // RING, ring_body, 0)

        for j in range(RING):
            wait_scatter(j, NCH - RING + j)

        plsc.subcore_barrier()

        pltpu.sync_copy(slab.at[stripe], out_hbm.at[ci, cc, si])
        plsc.subcore_barrier()
        return carry

    lax.fori_loop(0, CORE_NUM, core_phase, 0)


_sc_spmm = pl.kernel(
    _sc_spmm_body,
    out_type=jax.ShapeDtypeStruct((NC, CORE_NUM, NS, N // NS, D_OUT),
                                  jnp.float32),
    mesh=plsc.VectorSubcoreMesh(core_axis_name="c", subcore_axis_name="s",
                                num_cores=NC, num_subcores=NS),
    compiler_params=pltpu.CompilerParams(needs_layout_passes=False,
                                         use_tc_tiling_on_sc=False),
    scratch_types=[
        pltpu.VMEM((NCH, CHUNK), jnp.int32),
        pltpu.VMEM((NCH, CHUNK), jnp.int32),
        pltpu.VMEM((NCH, CHUNK), jnp.float32),
        [pltpu.VMEM((CHUNK, D_OUT), jnp.float32) for _ in range(RING)],
        [pltpu.VMEM((CHUNK, D_OUT), jnp.float32) for _ in range(RING)],
        [pltpu.SemaphoreType.DMA for _ in range(RING)],
        [pltpu.SemaphoreType.DMA for _ in range(RING)],
        pltpu.VMEM_SHARED((N, D_OUT), jnp.float32),
        pltpu.VMEM_SHARED((N, D_OUT), jnp.float32),
    ],
)


_BLK = 1000
_NBLK = N // _BLK


def _dotT(a, w):
    return lax.dot_general(a, w, (((1,), (1,)), ((), ())),
                           preferred_element_type=jnp.float32)


def _mlp_body(x_ref, w1_ref, b1_ref, w2_ref, b2_ref, o_ref):
    x = x_ref[0]
    h = _dotT(x, w1_ref[0]) + b1_ref[0]
    o_ref[0] = _dotT(h, w2_ref[0]) + b2_ref[0]


def _mlp(x_list, W1, b1, W2, b2):
    return pl.pallas_call(
        _mlp_body,
        grid=(DURATION, _NBLK),
        in_specs=[
            pl.BlockSpec((1, _BLK, D_IN), lambda t, i: (t, i, 0)),
            pl.BlockSpec((1, D_HID, D_IN), lambda t, i: (t, 0, 0)),
            pl.BlockSpec((1, 1, D_HID), lambda t, i: (t, 0, 0)),
            pl.BlockSpec((1, D_HID, D_HID), lambda t, i: (t, 0, 0)),
            pl.BlockSpec((1, 1, D_HID), lambda t, i: (t, 0, 0)),
        ],
        out_specs=pl.BlockSpec((1, _BLK, D_HID), lambda t, i: (t, i, 0)),
        out_shape=jax.ShapeDtypeStruct((DURATION, N, D_HID), jnp.float32),
    )(x_list, W1, b1.reshape(DURATION, 1, D_HID), W2,
      b2.reshape(DURATION, 1, D_HID))


def _layer_norm(x, g, b, eps=1e-5):
    mu = jnp.mean(x, axis=-1, keepdims=True)
    var = jnp.mean((x - mu) ** 2, axis=-1, keepdims=True)
    return (x - mu) / jnp.sqrt(var + eps) * g + b


def _gru_gates(x_t, h, Wih, Whh, bih, bhh):
    gi = _dotT(x_t, Wih) + bih
    gh = _dotT(h, Whh) + bhh
    r = jax.nn.sigmoid(gi[:, 0:D_OUT] + gh[:, 0:D_OUT])
    z = jax.nn.sigmoid(gi[:, D_OUT:2 * D_OUT] + gh[:, D_OUT:2 * D_OUT])
    n = jnp.tanh(gi[:, 2 * D_OUT:] + r * gh[:, 2 * D_OUT:])
    return (1.0 - z) * n + z * h


def _diff_body(p_ref, wih_ref, whh_ref, bih_ref, bhh_ref, g_ref, b_ref, o_ref):
    p = p_ref[...]
    Wih = wih_ref[...]
    Whh = whh_ref[...]
    bih = bih_ref[...]
    bhh = bhh_ref[...]
    hx = None
    h = jnp.zeros((_BLK, D_OUT), jnp.float32)
    acc = jnp.zeros((_BLK, D_OUT), jnp.float32)
    for c in range(CORE_NUM):
        sp = p[0, c] + p[1, c]
        hx = sp if hx is None else hx + sp
        h = _gru_gates(hx, h, Wih, Whh, bih, bhh)
        acc = acc + h
    o_ref[...] = _layer_norm(acc, g_ref[...], b_ref[...])


def _gru_diff(part, Wih, Whh, bih, bhh, g, b):
    return pl.pallas_call(
        _diff_body,
        grid=(_NBLK,),
        in_specs=[
            pl.BlockSpec((NC, CORE_NUM, _BLK, D_OUT), lambda i: (0, 0, i, 0)),
            pl.BlockSpec((H3, D_OUT), lambda i: (0, 0)),
            pl.BlockSpec((H3, D_OUT), lambda i: (0, 0)),
            pl.BlockSpec((1, H3), lambda i: (0, 0)),
            pl.BlockSpec((1, H3), lambda i: (0, 0)),
            pl.BlockSpec((1, D_OUT), lambda i: (0, 0)),
            pl.BlockSpec((1, D_OUT), lambda i: (0, 0)),
        ],
        out_specs=pl.BlockSpec((_BLK, D_OUT), lambda i: (i, 0)),
        out_shape=jax.ShapeDtypeStruct((N, D_OUT), jnp.float32),
    )(part, Wih, Whh, bih.reshape(1, H3), bhh.reshape(1, H3),
      g.reshape(1, D_OUT), b.reshape(1, D_OUT))


def _rnn_body(hx_ref, wih_ref, whh_ref, bih_ref, bhh_ref, g_ref, b_ref, o_ref):
    hx = hx_ref[...]
    Wih = wih_ref[...]
    Whh = whh_ref[...]
    bih = bih_ref[...]
    bhh = bhh_ref[...]
    g = g_ref[...]
    b = b_ref[...]
    h = jnp.zeros((_BLK, D_OUT), jnp.float32)
    for t in range(DURATION):
        h = _gru_gates(hx[t], h, Wih, Whh, bih, bhh)
        o_ref[t] = _layer_norm(h, g, b)


def _rnn_final(hx, Wih, Whh, bih, bhh, g, b):
    return pl.pallas_call(
        _rnn_body,
        grid=(_NBLK,),
        in_specs=[
            pl.BlockSpec((DURATION, _BLK, D_OUT), lambda i: (0, i, 0)),
            pl.BlockSpec((H3, D_OUT), lambda i: (0, 0)),
            pl.BlockSpec((H3, D_OUT), lambda i: (0, 0)),
            pl.BlockSpec((1, H3), lambda i: (0, 0)),
            pl.BlockSpec((1, H3), lambda i: (0, 0)),
            pl.BlockSpec((1, D_OUT), lambda i: (0, 0)),
            pl.BlockSpec((1, D_OUT), lambda i: (0, 0)),
        ],
        out_specs=pl.BlockSpec((DURATION, _BLK, D_OUT), lambda i: (0, i, 0)),
        out_shape=jax.ShapeDtypeStruct((DURATION, N, D_OUT), jnp.float32),
    )(hx, Wih, Whh, bih.reshape(1, H3), bhh.reshape(1, H3),
      g.reshape(1, D_OUT), b.reshape(1, D_OUT))


def kernel(x_list, edge_index, edge_weight, mlp_W1, mlp_b1, mlp_W2, mlp_b2,
           cd_Wih, cd_Whh, cd_bih, cd_bhh, cd_ln_g, cd_ln_b,
           rnn_Wih, rnn_Whh, rnn_bih, rnn_bhh, norm_g, norm_b):
    x0 = _mlp(x_list, mlp_W1, mlp_b1, mlp_W2, mlp_b2)

    ei32 = edge_index.astype(jnp.int32)
    pad = ((0, 0), (0, 0), (0, 0), (0, EPW_P - EPW))
    src_all = jnp.pad(ei32[:, :, 0, :].reshape(DURATION, CORE_NUM, NW, EPW),
                      pad).reshape(DURATION, CORE_NUM, NW, NCH, CHUNK)
    dst_all = jnp.pad(ei32[:, :, 1, :].reshape(DURATION, CORE_NUM, NW, EPW),
                      pad).reshape(DURATION, CORE_NUM, NW, NCH, CHUNK)
    w_all = jnp.pad(edge_weight.reshape(DURATION, CORE_NUM, NW, EPW),
                    pad).reshape(DURATION, CORE_NUM, NW, NCH, CHUNK)
    zeros625 = jnp.zeros((625, D_OUT), jnp.float32)

    outs = []
    for t in range(DURATION):
        x_cur = x0[t]
        for d in range(DIFF_NUM):
            part = _sc_spmm(x_cur, src_all[t], dst_all[t], w_all[t], zeros625)
            part = part.reshape(NC, CORE_NUM, N, D_OUT)
            x_cur = _gru_diff(part, cd_Wih[t, d], cd_Whh[t, d],
                              cd_bih[t, d], cd_bhh[t, d],
                              cd_ln_g[t, d], cd_ln_b[t, d])
        outs.append(x_cur)

    hx = jnp.stack(outs, axis=0)
    return _rnn_final(hx, rnn_Wih, rnn_Whh, rnn_bih, rnn_bhh, norm_g, norm_b)
